# Initial kernel scaffold; baseline (speedup 1.0000x reference)
import jax
import jax.numpy as jnp
from jax.experimental import pallas as pl


def kernel(token_ids, emb_matrix):
    raise NotImplementedError("write your pallas kernel here")


# speedup vs baseline: 4.5425x; 4.5425x over previous
import functools

import jax
import jax.numpy as jnp
from jax import lax
from jax.experimental import pallas as pl
from jax.experimental.pallas import tpu as pltpu
from jax.experimental.pallas import tpu_sc as plsc

EMBED_DIM = 64
CHUNK = 128


@functools.cache
def _build(B: int, V: int):
    info = plsc.get_sparse_core_info()
    nw = info.num_cores * info.num_subcores
    b_per_w = B // nw
    n_chunks = b_per_w // CHUNK
    assert b_per_w * nw == B and n_chunks * CHUNK == b_per_w and n_chunks % 2 == 0

    mesh = plsc.VectorSubcoreMesh(core_axis_name="c", subcore_axis_name="s")

    @functools.partial(
        pl.kernel,
        mesh=mesh,
        compiler_params=pltpu.CompilerParams(use_tc_tiling_on_sc=False),
        out_type=jax.ShapeDtypeStruct((B, EMBED_DIM), jnp.float32),
        scratch_types=[
            pltpu.VMEM((1, n_chunks, CHUNK), jnp.int32),
            pltpu.VMEM((CHUNK, EMBED_DIM), jnp.float32),
            pltpu.VMEM((CHUNK, EMBED_DIM), jnp.float32),
            pltpu.SemaphoreType.DMA,
            pltpu.SemaphoreType.DMA,
            pltpu.SemaphoreType.DMA,
            pltpu.SemaphoreType.DMA,
        ],
    )
    def gather_kernel(table_hbm, idx_hbm, out_hbm, idx_v, buf0, buf1,
                      g0, g1, w0, w1):
        wid = lax.axis_index("s") * info.num_cores + lax.axis_index("c")
        base = wid * b_per_w
        pltpu.sync_copy(idx_hbm.at[pl.ds(wid, 1)], idx_v)
        idx_rows = idx_v.at[0]

        bufs = (buf0, buf1)
        gsems = (g0, g1)
        wsems = (w0, w1)

        pltpu.async_copy(table_hbm.at[idx_rows.at[0]], buf0, g0)
        pltpu.async_copy(table_hbm.at[idx_rows.at[1]], buf1, g1)

        def step(t, _):
            for b in range(2):
                j = t * 2 + b
                buf, gsem, wsem = bufs[b], gsems[b], wsems[b]
                pltpu.make_async_copy(table_hbm.at[idx_rows.at[j]], buf, gsem).wait()
                out_slc = out_hbm.at[pl.ds(base + j * CHUNK, CHUNK)]
                pltpu.async_copy(buf, out_slc, wsem)
                pltpu.make_async_copy(buf, out_slc, wsem).wait()

                @pl.when(j + 2 < n_chunks)
                def _():
                    pltpu.async_copy(table_hbm.at[idx_rows.at[j + 2]], buf, gsem)

            return 0

        lax.fori_loop(0, n_chunks // 2, step, 0)

    return gather_kernel


def kernel(token_ids, emb_matrix):
    orig_shape = token_ids.shape
    flat = token_ids.reshape(-1).astype(jnp.int32)
    B = flat.shape[0]
    idx3d = flat.reshape(32, -1, CHUNK)
    out = _build(B, emb_matrix.shape[0])(emb_matrix, idx3d)
    return out.reshape(*orig_shape, EMBED_DIM)


# speedup vs baseline: 4.5902x; 1.0105x over previous
import functools

import jax
import jax.numpy as jnp
from jax import lax
from jax.experimental import pallas as pl
from jax.experimental.pallas import tpu as pltpu
from jax.experimental.pallas import tpu_sc as plsc

EMBED_DIM = 64
CHUNK = 64
NBUF = 4


@functools.cache
def _build(B: int, V: int):
    info = plsc.get_sparse_core_info()
    nw = info.num_cores * info.num_subcores
    b_per_w = B // nw
    n_chunks = b_per_w // CHUNK
    assert b_per_w * nw == B and n_chunks * CHUNK == b_per_w
    assert n_chunks % NBUF == 0 and n_chunks >= 2 * NBUF

    mesh = plsc.VectorSubcoreMesh(core_axis_name="c", subcore_axis_name="s")

    @functools.partial(
        pl.kernel,
        mesh=mesh,
        compiler_params=pltpu.CompilerParams(use_tc_tiling_on_sc=False),
        out_type=jax.ShapeDtypeStruct((B, EMBED_DIM), jnp.float32),
        scratch_types=[
            pltpu.VMEM((1, n_chunks, CHUNK), jnp.int32),
            pltpu.VMEM((NBUF, CHUNK, EMBED_DIM), jnp.float32),
        ]
        + [pltpu.SemaphoreType.DMA] * (2 * NBUF),
    )
    def gather_kernel(table_hbm, idx_hbm, out_hbm, idx_v, bufs, *sems):
        gsems = sems[:NBUF]
        wsems = sems[NBUF:]
        wid = lax.axis_index("s") * info.num_cores + lax.axis_index("c")
        base = wid * b_per_w
        pltpu.sync_copy(idx_hbm.at[pl.ds(wid, 1)], idx_v)
        idx_rows = idx_v.at[0]

        def out_slc(j):
            return out_hbm.at[pl.ds(base + j * CHUNK, CHUNK)]

        pltpu.async_copy(table_hbm.at[idx_rows.at[0]], bufs.at[0], gsems[0])
        pltpu.async_copy(table_hbm.at[idx_rows.at[1]], bufs.at[1], gsems[1])

        def step(t, _):
            for b in range(NBUF):
                j = t * NBUF + b
                b2 = (b + 2) % NBUF
                @pl.when(j >= 2)
                def _():
                    pltpu.make_async_copy(
                        bufs.at[b2], out_slc(j - 2), wsems[b2]).wait()

                @pl.when(j + 2 < n_chunks)
                def _():
                    pltpu.async_copy(
                        table_hbm.at[idx_rows.at[j + 2]], bufs.at[b2],
                        gsems[b2])

                pltpu.make_async_copy(
                    table_hbm.at[idx_rows.at[j]], bufs.at[b], gsems[b]).wait()
                pltpu.async_copy(bufs.at[b], out_slc(j), wsems[b])
            return 0

        lax.fori_loop(0, n_chunks // NBUF, step, 0)

        for j in (n_chunks - 2, n_chunks - 1):
            b = j % NBUF
            pltpu.make_async_copy(bufs.at[b], out_slc(j), wsems[b]).wait()

    return gather_kernel


def kernel(token_ids, emb_matrix):
    orig_shape = token_ids.shape
    flat = token_ids.reshape(-1).astype(jnp.int32)
    B = flat.shape[0]
    idx3d = flat.reshape(32, -1, CHUNK)
    out = _build(B, emb_matrix.shape[0])(emb_matrix, idx3d)
    return out.reshape(*orig_shape, EMBED_DIM)
